# Initial kernel scaffold; baseline (speedup 1.0000x reference)
import jax
import jax.numpy as jnp
from jax.experimental import pallas as pl


def kernel(loc_data, conf_data, prior_data):
    raise NotImplementedError("write your pallas kernel here")


# speedup vs baseline: 3.1334x; 3.1334x over previous
import functools

import jax
import jax.numpy as jnp
from jax import lax
from jax.experimental import pallas as pl
from jax.experimental.pallas import tpu as pltpu
from jax.experimental.pallas import tpu_sc as plsc

_C = 21
_K = 200
_CONF = 0.99
_NMS = 0.45
_P = 20000
_N = 4
_ROWS = _N * (_C - 1)
_L = 16
_SLOTS = 208
_NCH = _SLOTS // _L
_CAND = _P + 2 * _L
_NW = 32
_OUTW = 5 * _SLOTS


def _row_body(r, conf_hbm, loc_hbm, pri_hbm, out_hbm,
              scores, cand_s, cand_i, top_s,
              loc_rows, pri_rows, bx1, by1, bx2, by2, bar, keep, obuf, sem):
    lanes = lax.iota(jnp.int32, _L)
    ninf = jnp.full((_L,), -jnp.inf, jnp.float32)
    ones_i = jnp.full((_L,), 1, jnp.int32)
    zeros_i = jnp.zeros((_L,), jnp.int32)
    thr = jnp.full((_L,), _CONF, jnp.float32)
    true16 = lanes < jnp.full((_L,), _L, jnp.int32)
    img = r // (_C - 1)

    pltpu.sync_copy(conf_hbm.at[r], scores)

    def comp_body(c, w):
        s = scores[pl.ds(c * _L, _L)]
        m = s > thr
        cs = plsc.cumsum(jnp.where(m, ones_i, zeros_i))
        cnt = jnp.max(cs)

        @pl.when(cnt > 0)
        def _():
            pos = w + cs - 1
            plsc.store_scatter(cand_s, [pos], s, mask=m)
            plsc.store_scatter(cand_i, [pos], c * _L + lanes, mask=m)

        return w + cnt

    count = lax.fori_loop(0, _P // _L, comp_body, jnp.int32(0))

    plsc.store_scatter(cand_s, [count + lanes], ninf, mask=true16)
    plsc.store_scatter(cand_i, [count + lanes], zeros_i, mask=true16)
    nch = count // _L + 1

    plsc.store_scatter(top_s, [192 + lanes], ninf, mask=true16)

    lane0 = lanes == zeros_i

    def ext_body(slot, _):
        def scan_body(c, carry):
            bv, bc = carry
            v = cand_s[pl.ds(c * _L, _L)]
            upd = v > bv
            return (jnp.where(upd, v, bv),
                    jnp.where(upd, jnp.full((_L,), c, jnp.int32), bc))

        bv, bc = lax.fori_loop(0, nch, scan_body, (ninf, zeros_i))
        m = jnp.max(bv)
        cpos = bc * _L + lanes
        jstar = jnp.min(jnp.where(bv == jnp.full((_L,), m, jnp.float32),
                                  cpos, jnp.full((_L,), 2**30, jnp.int32)))
        cbase = (jstar // _L) * _L
        ich = cand_i[pl.ds(cbase, _L)]
        p = jnp.max(jnp.where(lanes == jnp.full((_L,), jstar - cbase,
                                                jnp.int32),
                              ich, jnp.full((_L,), -1, jnp.int32)))
        plsc.store_scatter(cand_s, [jnp.full((_L,), jstar, jnp.int32)],
                           ninf, mask=lane0)
        plsc.store_scatter(top_s, [jnp.full((_L,), slot, jnp.int32)],
                           jnp.full((_L,), m, jnp.float32), mask=lane0)
        pltpu.make_async_copy(pri_hbm.at[p], pri_rows.at[slot], sem).start()
        pltpu.make_async_copy(loc_hbm.at[p + img * _P],
                              loc_rows.at[slot], sem).start()
        return 0

    lax.fori_loop(0, _K, ext_body, 0)

    pltpu.make_async_copy(pri_hbm.at[pl.ds(0, _K)],
                          pri_rows.at[pl.ds(0, _K)], sem).wait()
    pltpu.make_async_copy(loc_hbm.at[pl.ds(0, _K)],
                          loc_rows.at[pl.ds(0, _K)], sem).wait()

    def dec_body(c, _):
        base = c * _L
        rowi = base + lanes
        z = zeros_i
        px = plsc.load_gather(pri_rows, [rowi, z])
        py = plsc.load_gather(pri_rows, [rowi, z + 1])
        pw = plsc.load_gather(pri_rows, [rowi, z + 2])
        ph = plsc.load_gather(pri_rows, [rowi, z + 3])
        lx = plsc.load_gather(loc_rows, [rowi, z])
        ly = plsc.load_gather(loc_rows, [rowi, z + 1])
        lw = plsc.load_gather(loc_rows, [rowi, z + 2])
        lh = plsc.load_gather(loc_rows, [rowi, z + 3])
        cx = px + lx * 0.1 * pw
        cy = py + ly * 0.1 * ph
        wb = pw * jnp.exp(lw * 0.2)
        hb = ph * jnp.exp(lh * 0.2)
        x1 = cx - wb / 2.0
        y1 = cy - hb / 2.0
        x2 = cx + wb / 2.0
        y2 = cy + hb / 2.0
        bx1[pl.ds(base, _L)] = x1
        by1[pl.ds(base, _L)] = y1
        bx2[pl.ds(base, _L)] = x2
        by2[pl.ds(base, _L)] = y2
        bar[pl.ds(base, _L)] = (x2 - x1) * (y2 - y1)
        s = top_s[pl.ds(base, _L)]
        keep[pl.ds(base, _L)] = jnp.where(s > thr, ones_i, zeros_i)
        return 0

    lax.fori_loop(0, _NCH, dec_body, 0)

    nms_v = jnp.full((_L,), _NMS, jnp.float32)

    def nms_body(i, _):
        cb = (i // _L) * _L
        ln = i - cb
        sel = lanes == jnp.full((_L,), ln, jnp.int32)
        ki = jnp.max(jnp.where(sel, keep[pl.ds(cb, _L)], zeros_i))

        @pl.when(ki > 0)
        def _():
            x1i = jnp.max(jnp.where(sel, bx1[pl.ds(cb, _L)], ninf))
            y1i = jnp.max(jnp.where(sel, by1[pl.ds(cb, _L)], ninf))
            x2i = jnp.max(jnp.where(sel, bx2[pl.ds(cb, _L)], ninf))
            y2i = jnp.max(jnp.where(sel, by2[pl.ds(cb, _L)], ninf))
            ai = (x2i - x1i) * (y2i - y1i)

            def sup_body(c, _):
                b = c * _L
                ltx = jnp.maximum(x1i, bx1[pl.ds(b, _L)])
                lty = jnp.maximum(y1i, by1[pl.ds(b, _L)])
                rbx = jnp.minimum(x2i, bx2[pl.ds(b, _L)])
                rby = jnp.minimum(y2i, by2[pl.ds(b, _L)])
                ww = jnp.maximum(rbx - ltx, 0.0)
                hh = jnp.maximum(rby - lty, 0.0)
                inter = ww * hh
                iou = inter / (ai + bar[pl.ds(b, _L)] - inter)
                sup = (iou > nms_v) & ((b + lanes) > i)
                kc = keep[pl.ds(b, _L)]
                keep[pl.ds(b, _L)] = jnp.where(sup, zeros_i, kc)
                return 0

            lax.fori_loop(0, _NCH, sup_body, 0)

        return 0

    lax.fori_loop(0, _K, nms_body, 0)

    def zero_body(c, _):
        obuf[pl.ds(c * _L, _L)] = jnp.zeros((_L,), jnp.float32)
        return 0

    lax.fori_loop(0, _OUTW // _L, zero_body, 0)

    def out_body(c, wk):
        b = c * _L
        k = keep[pl.ds(b, _L)] > zeros_i
        cs = plsc.cumsum(jnp.where(k, ones_i, zeros_i))
        cnt = jnp.max(cs)

        @pl.when(cnt > 0)
        def _():
            pos = wk + cs - 1
            plsc.store_scatter(obuf, [pos], top_s[pl.ds(b, _L)], mask=k)
            plsc.store_scatter(obuf, [pos + _SLOTS], bx1[pl.ds(b, _L)], mask=k)
            plsc.store_scatter(obuf, [pos + 2 * _SLOTS], by1[pl.ds(b, _L)],
                               mask=k)
            plsc.store_scatter(obuf, [pos + 3 * _SLOTS], bx2[pl.ds(b, _L)],
                               mask=k)
            plsc.store_scatter(obuf, [pos + 4 * _SLOTS], by2[pl.ds(b, _L)],
                               mask=k)

        return wk + cnt

    lax.fori_loop(0, _NCH, out_body, jnp.int32(0))

    pltpu.sync_copy(obuf, out_hbm.at[r])


@jax.jit
def _detect_sc(conf_rows, loc_flat, priors):
    mesh = plsc.VectorSubcoreMesh(core_axis_name="c", subcore_axis_name="s")

    @functools.partial(
        pl.kernel,
        out_type=jax.ShapeDtypeStruct((_ROWS, _OUTW), jnp.float32),
        mesh=mesh,
        compiler_params=pltpu.CompilerParams(needs_layout_passes=False,
                                             use_tc_tiling_on_sc=False),
        scratch_types=[
            pltpu.VMEM((_P,), jnp.float32),
            pltpu.VMEM((_CAND,), jnp.float32),
            pltpu.VMEM((_CAND,), jnp.int32),
            pltpu.VMEM((_SLOTS,), jnp.float32),
            pltpu.VMEM((_SLOTS, 4), jnp.float32),
            pltpu.VMEM((_SLOTS, 4), jnp.float32),
            pltpu.VMEM((_SLOTS,), jnp.float32),
            pltpu.VMEM((_SLOTS,), jnp.float32),
            pltpu.VMEM((_SLOTS,), jnp.float32),
            pltpu.VMEM((_SLOTS,), jnp.float32),
            pltpu.VMEM((_SLOTS,), jnp.float32),
            pltpu.VMEM((_SLOTS,), jnp.int32),
            pltpu.VMEM((_OUTW,), jnp.float32),
            pltpu.SemaphoreType.DMA,
        ],
    )
    def k(conf_hbm, loc_hbm, pri_hbm, out_hbm, *scr):
        wid = lax.axis_index("s") * 2 + lax.axis_index("c")
        for t in range(3):
            r = t * _NW + wid

            @pl.when(r < _ROWS)
            def _():
                _row_body(r, conf_hbm, loc_hbm, pri_hbm, out_hbm, *scr)

    return k(conf_rows, loc_flat, priors)


def kernel(loc_data, conf_data, prior_data):
    conf_rows = jnp.transpose(conf_data, (0, 2, 1))[:, 1:, :].reshape(
        _ROWS, _P)
    loc_flat = loc_data.reshape(_N * _P, 4)
    out = _detect_sc(conf_rows, loc_flat, prior_data)
    rows = out.reshape(_ROWS, 5, _SLOTS)[:, :, :_K]
    rows = jnp.transpose(rows, (0, 2, 1)).reshape(_N, _C - 1, _K, 5)
    zero = jnp.zeros((_N, 1, _K, 5), rows.dtype)
    return jnp.concatenate([zero, rows], axis=1)


# speedup vs baseline: 3.4241x; 1.0928x over previous
import functools

import jax
import jax.numpy as jnp
from jax import lax
from jax.experimental import pallas as pl
from jax.experimental.pallas import tpu as pltpu
from jax.experimental.pallas import tpu_sc as plsc

_C = 21
_K = 200
_CONF = 0.99
_NMS = 0.45
_P = 20000
_N = 4
_ROWS = _N * (_C - 1)
_L = 16
_SLOTS = 208
_NCH = _SLOTS // _L
_CAND = _P + 2 * _L
_NW = 32
_OUTW = 5 * _SLOTS


def _row_body(r, conf_hbm, loc_hbm, pri_hbm, out_hbm,
              scores, cand_s, cand_i, top_s,
              loc_rows, pri_rows, bx1, by1, bx2, by2, bar, keep, obuf, sem):
    lanes = lax.iota(jnp.int32, _L)
    ninf = jnp.full((_L,), -jnp.inf, jnp.float32)
    ones_i = jnp.full((_L,), 1, jnp.int32)
    zeros_i = jnp.zeros((_L,), jnp.int32)
    thr = jnp.full((_L,), _CONF, jnp.float32)
    true16 = lanes < jnp.full((_L,), _L, jnp.int32)
    img = r // (_C - 1)

    pltpu.sync_copy(conf_hbm.at[r], scores)

    def comp_body(c, w):
        s = scores[pl.ds(c * _L, _L)]
        m = s > thr
        cs = plsc.cumsum(jnp.where(m, ones_i, zeros_i))
        cnt = jnp.max(cs)

        @pl.when(cnt > 0)
        def _():
            pos = w + cs - 1
            plsc.store_scatter(cand_s, [pos], s, mask=m)
            plsc.store_scatter(cand_i, [pos], c * _L + lanes, mask=m)

        return w + cnt

    count = lax.fori_loop(0, _P // _L, comp_body, jnp.int32(0))

    plsc.store_scatter(cand_s, [count + lanes], ninf, mask=true16)
    plsc.store_scatter(cand_i, [count + lanes], zeros_i, mask=true16)
    nch = count // _L + 1

    plsc.store_scatter(top_s, [192 + lanes], ninf, mask=true16)

    lane0 = lanes == zeros_i

    def ext_body(slot, _):
        def scan_body(c, carry):
            bv, bc = carry
            v = cand_s[pl.ds(c * _L, _L)]
            upd = v > bv
            return (jnp.where(upd, v, bv),
                    jnp.where(upd, jnp.full((_L,), c, jnp.int32), bc))

        bv, bc = lax.fori_loop(0, nch, scan_body, (ninf, zeros_i))
        m = jnp.max(bv)
        cpos = bc * _L + lanes
        jstar = jnp.min(jnp.where(bv == jnp.full((_L,), m, jnp.float32),
                                  cpos, jnp.full((_L,), 2**30, jnp.int32)))
        cbase = (jstar // _L) * _L
        ich = cand_i[pl.ds(cbase, _L)]
        p = jnp.max(jnp.where(lanes == jnp.full((_L,), jstar - cbase,
                                                jnp.int32),
                              ich, jnp.full((_L,), -1, jnp.int32)))
        plsc.store_scatter(cand_s, [jnp.full((_L,), jstar, jnp.int32)],
                           ninf, mask=lane0)
        plsc.store_scatter(top_s, [jnp.full((_L,), slot, jnp.int32)],
                           jnp.full((_L,), m, jnp.float32), mask=lane0)
        pltpu.make_async_copy(pri_hbm.at[p], pri_rows.at[slot], sem).start()
        pltpu.make_async_copy(loc_hbm.at[p + img * _P],
                              loc_rows.at[slot], sem).start()
        return 0

    lax.fori_loop(0, _K, ext_body, 0)

    pltpu.make_async_copy(pri_hbm.at[pl.ds(0, _K)],
                          pri_rows.at[pl.ds(0, _K)], sem).wait()
    pltpu.make_async_copy(loc_hbm.at[pl.ds(0, _K)],
                          loc_rows.at[pl.ds(0, _K)], sem).wait()

    def dec_body(c, _):
        base = c * _L
        rowi = base + lanes
        z = zeros_i
        px = plsc.load_gather(pri_rows, [rowi, z])
        py = plsc.load_gather(pri_rows, [rowi, z + 1])
        pw = plsc.load_gather(pri_rows, [rowi, z + 2])
        ph = plsc.load_gather(pri_rows, [rowi, z + 3])
        lx = plsc.load_gather(loc_rows, [rowi, z])
        ly = plsc.load_gather(loc_rows, [rowi, z + 1])
        lw = plsc.load_gather(loc_rows, [rowi, z + 2])
        lh = plsc.load_gather(loc_rows, [rowi, z + 3])
        cx = px + lx * 0.1 * pw
        cy = py + ly * 0.1 * ph
        wb = pw * jnp.exp(lw * 0.2)
        hb = ph * jnp.exp(lh * 0.2)
        x1 = cx - wb / 2.0
        y1 = cy - hb / 2.0
        x2 = cx + wb / 2.0
        y2 = cy + hb / 2.0
        bx1[pl.ds(base, _L)] = x1
        by1[pl.ds(base, _L)] = y1
        bx2[pl.ds(base, _L)] = x2
        by2[pl.ds(base, _L)] = y2
        bar[pl.ds(base, _L)] = (x2 - x1) * (y2 - y1)
        s = top_s[pl.ds(base, _L)]
        keep[pl.ds(base, _L)] = jnp.where(s > thr, ones_i, zeros_i)
        return 0

    lax.fori_loop(0, _NCH, dec_body, 0)

    nms_v = jnp.full((_L,), _NMS, jnp.float32)

    def nms_body(i, _):
        cb = (i // _L) * _L
        ln = i - cb
        sel = lanes == jnp.full((_L,), ln, jnp.int32)
        ki = jnp.max(jnp.where(sel, keep[pl.ds(cb, _L)], zeros_i))

        @pl.when(ki > 0)
        def _():
            isplat = jnp.full((_L,), i, jnp.int32)
            x1i = plsc.load_gather(bx1, [isplat])
            y1i = plsc.load_gather(by1, [isplat])
            x2i = plsc.load_gather(bx2, [isplat])
            y2i = plsc.load_gather(by2, [isplat])
            ai = (x2i - x1i) * (y2i - y1i)

            def sup_body(c, _):
                b = c * _L
                ltx = jnp.maximum(x1i, bx1[pl.ds(b, _L)])
                lty = jnp.maximum(y1i, by1[pl.ds(b, _L)])
                rbx = jnp.minimum(x2i, bx2[pl.ds(b, _L)])
                rby = jnp.minimum(y2i, by2[pl.ds(b, _L)])
                ww = jnp.maximum(rbx - ltx, 0.0)
                hh = jnp.maximum(rby - lty, 0.0)
                inter = ww * hh
                iou = inter / (ai + bar[pl.ds(b, _L)] - inter)
                sup = (iou > nms_v) & ((b + lanes) > i)
                kc = keep[pl.ds(b, _L)]
                keep[pl.ds(b, _L)] = jnp.where(sup, zeros_i, kc)
                return 0

            lax.fori_loop(i // _L, _NCH, sup_body, 0)

        return 0

    lax.fori_loop(0, _K, nms_body, 0)

    def zero_body(c, _):
        obuf[pl.ds(c * _L, _L)] = jnp.zeros((_L,), jnp.float32)
        return 0

    lax.fori_loop(0, _OUTW // _L, zero_body, 0)

    def out_body(c, wk):
        b = c * _L
        k = keep[pl.ds(b, _L)] > zeros_i
        cs = plsc.cumsum(jnp.where(k, ones_i, zeros_i))
        cnt = jnp.max(cs)

        @pl.when(cnt > 0)
        def _():
            pos = wk + cs - 1
            plsc.store_scatter(obuf, [pos], top_s[pl.ds(b, _L)], mask=k)
            plsc.store_scatter(obuf, [pos + _SLOTS], bx1[pl.ds(b, _L)], mask=k)
            plsc.store_scatter(obuf, [pos + 2 * _SLOTS], by1[pl.ds(b, _L)],
                               mask=k)
            plsc.store_scatter(obuf, [pos + 3 * _SLOTS], bx2[pl.ds(b, _L)],
                               mask=k)
            plsc.store_scatter(obuf, [pos + 4 * _SLOTS], by2[pl.ds(b, _L)],
                               mask=k)

        return wk + cnt

    lax.fori_loop(0, _NCH, out_body, jnp.int32(0))

    pltpu.sync_copy(obuf, out_hbm.at[r])


@jax.jit
def _detect_sc(conf_rows, loc_flat, priors):
    mesh = plsc.VectorSubcoreMesh(core_axis_name="c", subcore_axis_name="s")

    @functools.partial(
        pl.kernel,
        out_type=jax.ShapeDtypeStruct((_ROWS, _OUTW), jnp.float32),
        mesh=mesh,
        compiler_params=pltpu.CompilerParams(needs_layout_passes=False,
                                             use_tc_tiling_on_sc=False),
        scratch_types=[
            pltpu.VMEM((_P,), jnp.float32),
            pltpu.VMEM((_CAND,), jnp.float32),
            pltpu.VMEM((_CAND,), jnp.int32),
            pltpu.VMEM((_SLOTS,), jnp.float32),
            pltpu.VMEM((_SLOTS, 4), jnp.float32),
            pltpu.VMEM((_SLOTS, 4), jnp.float32),
            pltpu.VMEM((_SLOTS,), jnp.float32),
            pltpu.VMEM((_SLOTS,), jnp.float32),
            pltpu.VMEM((_SLOTS,), jnp.float32),
            pltpu.VMEM((_SLOTS,), jnp.float32),
            pltpu.VMEM((_SLOTS,), jnp.float32),
            pltpu.VMEM((_SLOTS,), jnp.int32),
            pltpu.VMEM((_OUTW,), jnp.float32),
            pltpu.SemaphoreType.DMA,
        ],
    )
    def k(conf_hbm, loc_hbm, pri_hbm, out_hbm, *scr):
        wid = lax.axis_index("s") * 2 + lax.axis_index("c")
        for t in range(3):
            r = t * _NW + wid

            @pl.when(r < _ROWS)
            def _():
                _row_body(r, conf_hbm, loc_hbm, pri_hbm, out_hbm, *scr)

    return k(conf_rows, loc_flat, priors)


def kernel(loc_data, conf_data, prior_data):
    conf_rows = jnp.transpose(conf_data, (0, 2, 1))[:, 1:, :].reshape(
        _ROWS, _P)
    loc_flat = loc_data.reshape(_N * _P, 4)
    out = _detect_sc(conf_rows, loc_flat, prior_data)
    rows = out.reshape(_ROWS, 5, _SLOTS)[:, :, :_K]
    rows = jnp.transpose(rows, (0, 2, 1)).reshape(_N, _C - 1, _K, 5)
    zero = jnp.zeros((_N, 1, _K, 5), rows.dtype)
    return jnp.concatenate([zero, rows], axis=1)


# speedup vs baseline: 3.6320x; 1.0607x over previous
import functools

import jax
import jax.numpy as jnp
from jax import lax
from jax.experimental import pallas as pl
from jax.experimental.pallas import tpu as pltpu
from jax.experimental.pallas import tpu_sc as plsc

_C = 21
_K = 200
_CONF = 0.99
_NMS = 0.45
_P = 20000
_N = 4
_ROWS = _N * (_C - 1)
_L = 16
_SLOTS = 208
_NCH = _SLOTS // _L
_CAND = _P + 2 * _L
_NW = 32
_OUTW = 5 * _SLOTS


def _row_body(r, conf_hbm, loc_hbm, pri_hbm, out_hbm,
              scores, cand_s, cand_i, chm, top_s,
              loc_rows, pri_rows, bx1, by1, bx2, by2, bar, keep, obuf, sem):
    lanes = lax.iota(jnp.int32, _L)
    ninf = jnp.full((_L,), -jnp.inf, jnp.float32)
    ones_i = jnp.full((_L,), 1, jnp.int32)
    zeros_i = jnp.zeros((_L,), jnp.int32)
    thr = jnp.full((_L,), _CONF, jnp.float32)
    true16 = lanes < jnp.full((_L,), _L, jnp.int32)
    img = r // (_C - 1)

    pltpu.sync_copy(conf_hbm.at[r], scores)

    def comp_body(c, w):
        s = scores[pl.ds(c * _L, _L)]
        m = s > thr
        cs = plsc.cumsum(jnp.where(m, ones_i, zeros_i))
        cnt = jnp.max(cs)

        @pl.when(cnt > 0)
        def _():
            pos = w + cs - 1
            plsc.store_scatter(cand_s, [pos], s, mask=m)
            plsc.store_scatter(cand_i, [pos], c * _L + lanes, mask=m)

        return w + cnt

    count = lax.fori_loop(0, _P // _L, comp_body, jnp.int32(0))

    plsc.store_scatter(cand_s, [count + lanes], ninf, mask=true16)
    plsc.store_scatter(cand_i, [count + lanes], zeros_i, mask=true16)
    nch = count // _L + 1

    plsc.store_scatter(top_s, [192 + lanes], ninf, mask=true16)

    lane0 = lanes == zeros_i

    def chm_body(c, _):
        mc = jnp.max(cand_s[pl.ds(c * _L, _L)])
        plsc.store_scatter(chm, [jnp.full((_L,), c, jnp.int32)],
                           jnp.full((_L,), mc, jnp.float32), mask=lane0)
        return 0

    lax.fori_loop(0, nch, chm_body, 0)
    plsc.store_scatter(chm, [nch + lanes], ninf, mask=true16)
    mch = (nch + _L - 1) // _L

    big_i = jnp.full((_L,), 2**30, jnp.int32)

    def ext_body(slot, _):
        def scan_body(c, carry):
            bv, bc = carry
            v = chm[pl.ds(c * _L, _L)]
            upd = v > bv
            return (jnp.where(upd, v, bv),
                    jnp.where(upd, jnp.full((_L,), c, jnp.int32), bc))

        bv, bc = lax.fori_loop(0, mch, scan_body, (ninf, zeros_i))
        m = jnp.max(bv)
        msplat = jnp.full((_L,), m, jnp.float32)
        cstar = jnp.min(jnp.where(bv == msplat, bc * _L + lanes, big_i))
        v = cand_s[pl.ds(cstar * _L, _L)]
        lstar = jnp.min(jnp.where(v == msplat, lanes, big_i))
        jstar = cstar * _L + lstar
        p = jnp.max(plsc.load_gather(cand_i,
                                     [jnp.full((_L,), jstar, jnp.int32)]))
        plsc.store_scatter(cand_s, [jnp.full((_L,), jstar, jnp.int32)],
                           ninf, mask=lane0)
        m2 = jnp.max(jnp.where(lanes == jnp.full((_L,), lstar, jnp.int32),
                               ninf, v))
        plsc.store_scatter(chm, [jnp.full((_L,), cstar, jnp.int32)],
                           jnp.full((_L,), m2, jnp.float32), mask=lane0)
        plsc.store_scatter(top_s, [jnp.full((_L,), slot, jnp.int32)],
                           msplat, mask=lane0)
        pltpu.make_async_copy(pri_hbm.at[p], pri_rows.at[slot], sem).start()
        pltpu.make_async_copy(loc_hbm.at[p + img * _P],
                              loc_rows.at[slot], sem).start()
        return 0

    lax.fori_loop(0, _K, ext_body, 0)

    pltpu.make_async_copy(pri_hbm.at[pl.ds(0, _K)],
                          pri_rows.at[pl.ds(0, _K)], sem).wait()
    pltpu.make_async_copy(loc_hbm.at[pl.ds(0, _K)],
                          loc_rows.at[pl.ds(0, _K)], sem).wait()

    def dec_body(c, _):
        base = c * _L
        rowi = base + lanes
        z = zeros_i
        px = plsc.load_gather(pri_rows, [rowi, z])
        py = plsc.load_gather(pri_rows, [rowi, z + 1])
        pw = plsc.load_gather(pri_rows, [rowi, z + 2])
        ph = plsc.load_gather(pri_rows, [rowi, z + 3])
        lx = plsc.load_gather(loc_rows, [rowi, z])
        ly = plsc.load_gather(loc_rows, [rowi, z + 1])
        lw = plsc.load_gather(loc_rows, [rowi, z + 2])
        lh = plsc.load_gather(loc_rows, [rowi, z + 3])
        cx = px + lx * 0.1 * pw
        cy = py + ly * 0.1 * ph
        wb = pw * jnp.exp(lw * 0.2)
        hb = ph * jnp.exp(lh * 0.2)
        x1 = cx - wb / 2.0
        y1 = cy - hb / 2.0
        x2 = cx + wb / 2.0
        y2 = cy + hb / 2.0
        bx1[pl.ds(base, _L)] = x1
        by1[pl.ds(base, _L)] = y1
        bx2[pl.ds(base, _L)] = x2
        by2[pl.ds(base, _L)] = y2
        bar[pl.ds(base, _L)] = (x2 - x1) * (y2 - y1)
        s = top_s[pl.ds(base, _L)]
        keep[pl.ds(base, _L)] = jnp.where(s > thr, ones_i, zeros_i)
        return 0

    lax.fori_loop(0, _NCH, dec_body, 0)

    nms_v = jnp.full((_L,), _NMS, jnp.float32)

    def nms_body(i, _):
        cb = (i // _L) * _L
        ln = i - cb
        sel = lanes == jnp.full((_L,), ln, jnp.int32)
        ki = jnp.max(jnp.where(sel, keep[pl.ds(cb, _L)], zeros_i))

        @pl.when(ki > 0)
        def _():
            isplat = jnp.full((_L,), i, jnp.int32)
            x1i = plsc.load_gather(bx1, [isplat])
            y1i = plsc.load_gather(by1, [isplat])
            x2i = plsc.load_gather(bx2, [isplat])
            y2i = plsc.load_gather(by2, [isplat])
            ai = (x2i - x1i) * (y2i - y1i)

            def sup_body(c, _):
                b = c * _L
                ltx = jnp.maximum(x1i, bx1[pl.ds(b, _L)])
                lty = jnp.maximum(y1i, by1[pl.ds(b, _L)])
                rbx = jnp.minimum(x2i, bx2[pl.ds(b, _L)])
                rby = jnp.minimum(y2i, by2[pl.ds(b, _L)])
                ww = jnp.maximum(rbx - ltx, 0.0)
                hh = jnp.maximum(rby - lty, 0.0)
                inter = ww * hh
                iou = inter / (ai + bar[pl.ds(b, _L)] - inter)
                sup = (iou > nms_v) & ((b + lanes) > i)
                kc = keep[pl.ds(b, _L)]
                keep[pl.ds(b, _L)] = jnp.where(sup, zeros_i, kc)
                return 0

            lax.fori_loop(i // _L, _NCH, sup_body, 0)

        return 0

    lax.fori_loop(0, _K, nms_body, 0)

    def zero_body(c, _):
        obuf[pl.ds(c * _L, _L)] = jnp.zeros((_L,), jnp.float32)
        return 0

    lax.fori_loop(0, _OUTW // _L, zero_body, 0)

    def out_body(c, wk):
        b = c * _L
        k = keep[pl.ds(b, _L)] > zeros_i
        cs = plsc.cumsum(jnp.where(k, ones_i, zeros_i))
        cnt = jnp.max(cs)

        @pl.when(cnt > 0)
        def _():
            pos = wk + cs - 1
            plsc.store_scatter(obuf, [pos], top_s[pl.ds(b, _L)], mask=k)
            plsc.store_scatter(obuf, [pos + _SLOTS], bx1[pl.ds(b, _L)], mask=k)
            plsc.store_scatter(obuf, [pos + 2 * _SLOTS], by1[pl.ds(b, _L)],
                               mask=k)
            plsc.store_scatter(obuf, [pos + 3 * _SLOTS], bx2[pl.ds(b, _L)],
                               mask=k)
            plsc.store_scatter(obuf, [pos + 4 * _SLOTS], by2[pl.ds(b, _L)],
                               mask=k)

        return wk + cnt

    lax.fori_loop(0, _NCH, out_body, jnp.int32(0))

    pltpu.sync_copy(obuf, out_hbm.at[r])


@jax.jit
def _detect_sc(conf_rows, loc_flat, priors):
    mesh = plsc.VectorSubcoreMesh(core_axis_name="c", subcore_axis_name="s")

    @functools.partial(
        pl.kernel,
        out_type=jax.ShapeDtypeStruct((_ROWS, _OUTW), jnp.float32),
        mesh=mesh,
        compiler_params=pltpu.CompilerParams(needs_layout_passes=False,
                                             use_tc_tiling_on_sc=False),
        scratch_types=[
            pltpu.VMEM((_P,), jnp.float32),
            pltpu.VMEM((_CAND,), jnp.float32),
            pltpu.VMEM((_CAND,), jnp.int32),
            pltpu.VMEM((_CAND // _L + 2 * _L,), jnp.float32),
            pltpu.VMEM((_SLOTS,), jnp.float32),
            pltpu.VMEM((_SLOTS, 4), jnp.float32),
            pltpu.VMEM((_SLOTS, 4), jnp.float32),
            pltpu.VMEM((_SLOTS,), jnp.float32),
            pltpu.VMEM((_SLOTS,), jnp.float32),
            pltpu.VMEM((_SLOTS,), jnp.float32),
            pltpu.VMEM((_SLOTS,), jnp.float32),
            pltpu.VMEM((_SLOTS,), jnp.float32),
            pltpu.VMEM((_SLOTS,), jnp.int32),
            pltpu.VMEM((_OUTW,), jnp.float32),
            pltpu.SemaphoreType.DMA,
        ],
    )
    def k(conf_hbm, loc_hbm, pri_hbm, out_hbm, *scr):
        wid = lax.axis_index("s") * 2 + lax.axis_index("c")
        for t in range(3):
            r = t * _NW + wid

            @pl.when(r < _ROWS)
            def _():
                _row_body(r, conf_hbm, loc_hbm, pri_hbm, out_hbm, *scr)

    return k(conf_rows, loc_flat, priors)


def kernel(loc_data, conf_data, prior_data):
    conf_rows = jnp.transpose(conf_data, (0, 2, 1))[:, 1:, :].reshape(
        _ROWS, _P)
    loc_flat = loc_data.reshape(_N * _P, 4)
    out = _detect_sc(conf_rows, loc_flat, prior_data)
    rows = out.reshape(_ROWS, 5, _SLOTS)[:, :, :_K]
    rows = jnp.transpose(rows, (0, 2, 1)).reshape(_N, _C - 1, _K, 5)
    zero = jnp.zeros((_N, 1, _K, 5), rows.dtype)
    return jnp.concatenate([zero, rows], axis=1)


# speedup vs baseline: 3.8516x; 1.0605x over previous
import functools

import jax
import jax.numpy as jnp
from jax import lax
from jax.experimental import pallas as pl
from jax.experimental.pallas import tpu as pltpu
from jax.experimental.pallas import tpu_sc as plsc

_C = 21
_K = 200
_CONF = 0.99
_NMS = 0.45
_P = 20000
_N = 4
_ROWS = _N * (_C - 1)
_L = 16
_SLOTS = 208
_NCH = _SLOTS // _L
_CAND = _P + 2 * _L
_NW = 32
_OUTW = 5 * _SLOTS


def _row_body(r, conf_hbm, loc_hbm, pri_hbm, out_hbm,
              scores, cand_s, cand_i, chm, top_s,
              loc_rows, pri_rows, bx1, by1, bx2, by2, bar, keep, obuf, sem):
    lanes = lax.iota(jnp.int32, _L)
    ninf = jnp.full((_L,), -jnp.inf, jnp.float32)
    ones_i = jnp.full((_L,), 1, jnp.int32)
    zeros_i = jnp.zeros((_L,), jnp.int32)
    thr = jnp.full((_L,), _CONF, jnp.float32)
    true16 = lanes < jnp.full((_L,), _L, jnp.int32)
    img = r // (_C - 1)

    pltpu.sync_copy(conf_hbm.at[r], scores)

    def comp_body(c, w):
        s = scores[pl.ds(c * _L, _L)]
        m = s > thr
        cnt = plsc.all_reduce_population_count(m)[0]

        @pl.when(cnt > 0)
        def _():
            cs = plsc.cumsum(jnp.where(m, ones_i, zeros_i))
            pos = w + cs - 1
            plsc.store_scatter(cand_s, [pos], s, mask=m)
            plsc.store_scatter(cand_i, [pos], c * _L + lanes, mask=m)

        return w + cnt

    count = lax.fori_loop(0, _P // _L, comp_body, jnp.int32(0))

    plsc.store_scatter(cand_s, [count + lanes], ninf, mask=true16)
    plsc.store_scatter(cand_i, [count + lanes], zeros_i, mask=true16)
    nch = count // _L + 1

    plsc.store_scatter(top_s, [192 + lanes], ninf, mask=true16)

    lane0 = lanes == zeros_i

    def chm_body(c, _):
        mc = jnp.max(cand_s[pl.ds(c * _L, _L)])
        plsc.store_scatter(chm, [jnp.full((_L,), c, jnp.int32)],
                           jnp.full((_L,), mc, jnp.float32), mask=lane0)
        return 0

    lax.fori_loop(0, nch, chm_body, 0)
    plsc.store_scatter(chm, [nch + lanes], ninf, mask=true16)
    mch = (nch + _L - 1) // _L

    big_i = jnp.full((_L,), 2**30, jnp.int32)

    def ext_body(slot, _):
        def scan_body(c, carry):
            bv, bc = carry
            v = chm[pl.ds(c * _L, _L)]
            upd = v > bv
            return (jnp.where(upd, v, bv),
                    jnp.where(upd, jnp.full((_L,), c, jnp.int32), bc))

        bv, bc = lax.fori_loop(0, mch, scan_body, (ninf, zeros_i))
        m = jnp.max(bv)
        msplat = jnp.full((_L,), m, jnp.float32)
        cstar = jnp.min(jnp.where(bv == msplat, bc * _L + lanes, big_i))
        v = cand_s[pl.ds(cstar * _L, _L)]
        lstar = jnp.min(jnp.where(v == msplat, lanes, big_i))
        jstar = cstar * _L + lstar
        p = jnp.max(plsc.load_gather(cand_i,
                                     [jnp.full((_L,), jstar, jnp.int32)]))
        plsc.store_scatter(cand_s, [jnp.full((_L,), jstar, jnp.int32)],
                           ninf, mask=lane0)
        m2 = jnp.max(jnp.where(lanes == jnp.full((_L,), lstar, jnp.int32),
                               ninf, v))
        plsc.store_scatter(chm, [jnp.full((_L,), cstar, jnp.int32)],
                           jnp.full((_L,), m2, jnp.float32), mask=lane0)
        plsc.store_scatter(top_s, [jnp.full((_L,), slot, jnp.int32)],
                           msplat, mask=lane0)
        pltpu.make_async_copy(pri_hbm.at[p], pri_rows.at[slot], sem).start()
        pltpu.make_async_copy(loc_hbm.at[p + img * _P],
                              loc_rows.at[slot], sem).start()
        return 0

    lax.fori_loop(0, _K, ext_body, 0)

    pltpu.make_async_copy(pri_hbm.at[pl.ds(0, _K)],
                          pri_rows.at[pl.ds(0, _K)], sem).wait()
    pltpu.make_async_copy(loc_hbm.at[pl.ds(0, _K)],
                          loc_rows.at[pl.ds(0, _K)], sem).wait()

    def dec_body(c, _):
        base = c * _L
        rowi = base + lanes
        z = zeros_i
        px = plsc.load_gather(pri_rows, [rowi, z])
        py = plsc.load_gather(pri_rows, [rowi, z + 1])
        pw = plsc.load_gather(pri_rows, [rowi, z + 2])
        ph = plsc.load_gather(pri_rows, [rowi, z + 3])
        lx = plsc.load_gather(loc_rows, [rowi, z])
        ly = plsc.load_gather(loc_rows, [rowi, z + 1])
        lw = plsc.load_gather(loc_rows, [rowi, z + 2])
        lh = plsc.load_gather(loc_rows, [rowi, z + 3])
        cx = px + lx * 0.1 * pw
        cy = py + ly * 0.1 * ph
        wb = pw * jnp.exp(lw * 0.2)
        hb = ph * jnp.exp(lh * 0.2)
        x1 = cx - wb / 2.0
        y1 = cy - hb / 2.0
        x2 = cx + wb / 2.0
        y2 = cy + hb / 2.0
        bx1[pl.ds(base, _L)] = x1
        by1[pl.ds(base, _L)] = y1
        bx2[pl.ds(base, _L)] = x2
        by2[pl.ds(base, _L)] = y2
        bar[pl.ds(base, _L)] = (x2 - x1) * (y2 - y1)
        s = top_s[pl.ds(base, _L)]
        keep[pl.ds(base, _L)] = jnp.where(s > thr, ones_i, zeros_i)
        return 0

    lax.fori_loop(0, _NCH, dec_body, 0)

    nms_v = jnp.full((_L,), _NMS, jnp.float32)

    def nms_body(i, _):
        cb = (i // _L) * _L
        ln = i - cb
        sel = lanes == jnp.full((_L,), ln, jnp.int32)
        ki = jnp.max(jnp.where(sel, keep[pl.ds(cb, _L)], zeros_i))

        @pl.when(ki > 0)
        def _():
            isplat = jnp.full((_L,), i, jnp.int32)
            x1i = plsc.load_gather(bx1, [isplat])
            y1i = plsc.load_gather(by1, [isplat])
            x2i = plsc.load_gather(bx2, [isplat])
            y2i = plsc.load_gather(by2, [isplat])
            ai = (x2i - x1i) * (y2i - y1i)

            def sup_body(c, _):
                b = c * _L
                ltx = jnp.maximum(x1i, bx1[pl.ds(b, _L)])
                lty = jnp.maximum(y1i, by1[pl.ds(b, _L)])
                rbx = jnp.minimum(x2i, bx2[pl.ds(b, _L)])
                rby = jnp.minimum(y2i, by2[pl.ds(b, _L)])
                ww = jnp.maximum(rbx - ltx, 0.0)
                hh = jnp.maximum(rby - lty, 0.0)
                inter = ww * hh
                iou = inter / (ai + bar[pl.ds(b, _L)] - inter)
                sup = (iou > nms_v) & ((b + lanes) > i)
                kc = keep[pl.ds(b, _L)]
                keep[pl.ds(b, _L)] = jnp.where(sup, zeros_i, kc)
                return 0

            lax.fori_loop(i // _L, _NCH, sup_body, 0)

        return 0

    lax.fori_loop(0, _K, nms_body, 0)

    def zero_body(c, _):
        obuf[pl.ds(c * _L, _L)] = jnp.zeros((_L,), jnp.float32)
        return 0

    lax.fori_loop(0, _OUTW // _L, zero_body, 0)

    def out_body(c, wk):
        b = c * _L
        k = keep[pl.ds(b, _L)] > zeros_i
        cs = plsc.cumsum(jnp.where(k, ones_i, zeros_i))
        cnt = jnp.max(cs)

        @pl.when(cnt > 0)
        def _():
            pos = wk + cs - 1
            plsc.store_scatter(obuf, [pos], top_s[pl.ds(b, _L)], mask=k)
            plsc.store_scatter(obuf, [pos + _SLOTS], bx1[pl.ds(b, _L)], mask=k)
            plsc.store_scatter(obuf, [pos + 2 * _SLOTS], by1[pl.ds(b, _L)],
                               mask=k)
            plsc.store_scatter(obuf, [pos + 3 * _SLOTS], bx2[pl.ds(b, _L)],
                               mask=k)
            plsc.store_scatter(obuf, [pos + 4 * _SLOTS], by2[pl.ds(b, _L)],
                               mask=k)

        return wk + cnt

    lax.fori_loop(0, _NCH, out_body, jnp.int32(0))

    pltpu.sync_copy(obuf, out_hbm.at[r])


@jax.jit
def _detect_sc(conf_rows, loc_flat, priors):
    mesh = plsc.VectorSubcoreMesh(core_axis_name="c", subcore_axis_name="s")

    @functools.partial(
        pl.kernel,
        out_type=jax.ShapeDtypeStruct((_ROWS, _OUTW), jnp.float32),
        mesh=mesh,
        compiler_params=pltpu.CompilerParams(needs_layout_passes=False,
                                             use_tc_tiling_on_sc=False),
        scratch_types=[
            pltpu.VMEM((_P,), jnp.float32),
            pltpu.VMEM((_CAND,), jnp.float32),
            pltpu.VMEM((_CAND,), jnp.int32),
            pltpu.VMEM((_CAND // _L + 2 * _L,), jnp.float32),
            pltpu.VMEM((_SLOTS,), jnp.float32),
            pltpu.VMEM((_SLOTS, 4), jnp.float32),
            pltpu.VMEM((_SLOTS, 4), jnp.float32),
            pltpu.VMEM((_SLOTS,), jnp.float32),
            pltpu.VMEM((_SLOTS,), jnp.float32),
            pltpu.VMEM((_SLOTS,), jnp.float32),
            pltpu.VMEM((_SLOTS,), jnp.float32),
            pltpu.VMEM((_SLOTS,), jnp.float32),
            pltpu.VMEM((_SLOTS,), jnp.int32),
            pltpu.VMEM((_OUTW,), jnp.float32),
            pltpu.SemaphoreType.DMA,
        ],
    )
    def k(conf_hbm, loc_hbm, pri_hbm, out_hbm, *scr):
        wid = lax.axis_index("s") * 2 + lax.axis_index("c")
        for t in range(3):
            r = t * _NW + wid

            @pl.when(r < _ROWS)
            def _():
                _row_body(r, conf_hbm, loc_hbm, pri_hbm, out_hbm, *scr)

    return k(conf_rows, loc_flat, priors)


def kernel(loc_data, conf_data, prior_data):
    conf_rows = jnp.transpose(conf_data, (0, 2, 1))[:, 1:, :].reshape(
        _ROWS, _P)
    loc_flat = loc_data.reshape(_N * _P, 4)
    out = _detect_sc(conf_rows, loc_flat, prior_data)
    rows = out.reshape(_ROWS, 5, _SLOTS)[:, :, :_K]
    rows = jnp.transpose(rows, (0, 2, 1)).reshape(_N, _C - 1, _K, 5)
    zero = jnp.zeros((_N, 1, _K, 5), rows.dtype)
    return jnp.concatenate([zero, rows], axis=1)
